# Initial kernel scaffold; baseline (speedup 1.0000x reference)
import jax
import jax.numpy as jnp
from jax.experimental import pallas as pl


def kernel(pred, target):
    raise NotImplementedError("write your pallas kernel here")


# speedup vs baseline: 45.5814x; 45.5814x over previous
import functools
import math

import jax
import jax.numpy as jnp
import numpy as np
from jax import lax
from jax.experimental import pallas as pl
from jax.experimental.pallas import tpu as pltpu
from jax.experimental.pallas import tpu_sc as plsc

_NLAT = 128
_NLON = 256
_LMAX = 50
_MMAX = 50
_M = _NLAT * _NLON
_N = 2048

_PI32 = np.float32(np.pi)
_INV128 = np.float32(1.0 / 128.0)


def _cc_quad(n):
    nn = n - 1
    theta = np.pi * np.arange(0, n) / nn
    x = np.cos(theta)
    w = np.zeros(n)
    ii = np.arange(1, nn)
    v = np.ones(nn - 1)
    if nn % 2 == 0:
        w[0] = 1.0 / (nn ** 2 - 1)
        w[nn] = w[0]
        for k in range(1, nn // 2):
            v -= 2.0 * np.cos(2 * k * theta[ii]) / (4 * k ** 2 - 1)
        v -= np.cos(nn * theta[ii]) / (nn ** 2 - 1)
    else:
        w[0] = 1.0 / nn ** 2
        w[nn] = w[0]
        for k in range(1, (nn - 1) // 2 + 1):
            v -= 2.0 * np.cos(2 * k * theta[ii]) / (4 * k ** 2 - 1)
    w[ii] = 2.0 * v / nn
    return x, w


def _legendre(mmax, lmax, cost):
    nlat = cost.shape[0]
    sint = np.sqrt(np.clip(1.0 - cost ** 2, 0.0, None))
    p = np.zeros((mmax, lmax, nlat))
    p[0, 0] = np.sqrt(1.0 / (4.0 * np.pi))
    for m in range(1, mmax):
        p[m, m] = -np.sqrt((2.0 * m + 1.0) / (2.0 * m)) * sint * p[m - 1, m - 1]
    for m in range(0, mmax):
        if m + 1 < lmax:
            p[m, m + 1] = np.sqrt(2.0 * m + 3.0) * cost * p[m, m]
    for m in range(0, mmax):
        for l in range(m + 2, lmax):
            a = np.sqrt((4.0 * l * l - 1.0) / (l * l - m * m))
            b = np.sqrt(((2.0 * l + 1.0) * ((l - 1.0) ** 2 - m * m))
                        / ((2.0 * l - 3.0) * (l * l - m * m)))
            p[m, l] = a * cost * p[m, l - 1] - b * p[m, l - 2]
    return p


_cost_np, _wq_np = _cc_quad(_NLAT)
_W_MLK = (_legendre(_MMAX, _LMAX, _cost_np) * _wq_np[None, None, :])

_s2 = float(_LMAX ** 2)
_rw_np = np.exp(-(_LMAX - np.arange(1, _LMAX + 1)) ** 2 / (2.0 * _s2))

_WF_NP = _W_MLK.reshape(_MMAX * _LMAX, _NLAT).astype(np.float32)
_RWF_NP = np.tile(_rw_np.astype(np.float32), _MMAX)[:, None]

_jj = np.arange(_NLON)[:, None]
_mm = np.arange(_MMAX)[None, :]
_COSB_NP = (np.cos(2.0 * np.pi * _jj * _mm / _NLON)
            * (2.0 * np.pi / _NLON)).astype(np.float32)

_WF = jnp.asarray(_WF_NP)
_RWF = jnp.asarray(_RWF_NP)
_COSB = jnp.asarray(_COSB_NP)

import ml_dtypes as _mld

_gx_np = ((np.arange(_NLAT).astype(np.float32) / _NLAT) * np.float32(np.pi))
_gy_np = (((np.arange(_NLON).astype(np.float32) - _NLAT) / _NLAT)
          * np.float32(np.pi))
_GCONST_NP = np.zeros((2, _NLON), np.float32)
_GCONST_NP[0, :_NLAT] = _gx_np.astype(_mld.bfloat16).astype(np.float32)
_GCONST_NP[1, :] = _gy_np.astype(_mld.bfloat16).astype(np.float32)
_GCONST = jnp.asarray(_GCONST_NP)


def _to_spherical(coords):
    n = coords.shape[-1]
    r = jnp.linalg.norm(coords, axis=-1, keepdims=True)
    expanded = jnp.broadcast_to(jnp.flip(coords, -1)[..., None, :],
                                coords.shape + (n,))
    phi_norms = jnp.flip(jnp.linalg.norm(jnp.tril(expanded), axis=-1), -1)
    phi = jnp.arccos(jnp.clip(coords[..., :-2] / phi_norms[..., :-2], -1.0, 1.0))
    a = jnp.arccos(jnp.clip(coords[..., -2:-1] / phi_norms[..., -2:-1],
                            -1.0, 1.0))
    phi_final = a + (2.0 * math.pi - 2.0 * a) * (coords[..., -1:] < 0)
    return jnp.transpose(r, (0, 2, 1)), jnp.concatenate([phi, phi_final], -1)


_PPW = _M // 16
_GV = 4
_CH = 16 * _GV
_NCHUNK = _PPW // _CH


_VELT = np.float32(65537.0)


def _round_bf16(x):
    c = x * _VELT
    return c - (c - x)


_EPS = np.float32(1e-4)


def _sc_knn_body(cand_hbm, out_hbm, cand_v, res_v, tree_v):
    cloud = lax.axis_index("c")
    chunk = lax.axis_index("s")
    pltpu.sync_copy(cand_hbm.at[cloud], cand_v)

    def xreduce(v, neutral, op):
        tree_v[pl.ds(16, 16)] = jnp.full((16,), neutral, jnp.float32)
        r = v
        for sh in (8, 4, 2, 1):
            tree_v[pl.ds(0, 16)] = r
            w = tree_v[pl.ds(sh, 16)]
            r = op(r, w)
        return r[0]

    def chunk_body(ci, _):
        row = chunk * 8 + ci // 4
        lonb = (ci % 4) * _CH
        gxf = (row.astype(jnp.float32) * _INV128) * _PI32
        lane = lax.iota(jnp.int32, 16)
        gy = [((lane + (lonb + v * 16 - 128)).astype(jnp.float32)
               * _INV128) * _PI32 for v in range(_GV)]
        gu = [gxf * gxf + gy[v] * gy[v] for v in range(_GV)]
        gxb = _round_bf16(gxf)
        gx2 = gxb + gxb
        gyb = [_round_bf16(gy[v]) for v in range(_GV)]
        gub = [gxb * gxb + gyb[v] * gyb[v] for v in range(_GV)]
        mu = [gu[v] - gub[v] for v in range(_GV)]

        big = jnp.full((16,), 3.0e38, jnp.float32)
        zero = jnp.zeros((16,), jnp.float32)
        etamin = cand_v[4, pl.ds(0, 16)][0]

        def cnt_a(q, acc):
            p = cand_v[0, pl.ds(q * 16, 16)]
            return acc + jnp.where(p < gxb, 1.0, 0.0)

        c0f = lax.fori_loop(0, _N // 16, cnt_a, zero, unroll=8)
        c0 = xreduce(c0f, 0.0, jnp.add)
        b0 = jnp.clip(c0.astype(jnp.int32) // 16, 0, _N // 16 - 1)

        e1 = [big] * _GV
        e2 = [big] * _GV
        e3 = [big] * _GV
        for off in (-1, 0, 1):
            base1 = jnp.clip(b0 + off, 0, _N // 16 - 1) * 16
            p16 = cand_v[0, pl.ds(base1, 16)]
            q16 = cand_v[1, pl.ds(base1, 16)]
            w16 = cand_v[2, pl.ds(base1, 16)]
            for j in range(16):
                px = p16[j]
                py = q16[j]
                k2 = w16[j]
                s = k2 - gx2 * px
                t = py + py
                for v in range(_GV):
                    d = (gub[v] + s) - t * gyb[v]
                    c1 = d < e1[v]
                    c2 = d < e2[v]
                    c3 = d < e3[v]
                    e3[v] = jnp.where(c3, jnp.where(c2, e2[v], d), e3[v])
                    e2[v] = jnp.where(c2, jnp.where(c1, e1[v], d), e2[v])
                    e1[v] = jnp.where(c1, d, e1[v])
        m = jnp.maximum(jnp.maximum(e3[0], e3[1]),
                        jnp.maximum(e3[2], e3[3]))
        k3ub = xreduce(m, -3.0e38, jnp.maximum)
        thr = (k3ub - etamin) + _EPS

        def cnt_b(q, acc):
            lo_a, hi_a = acc
            p = cand_v[0, pl.ds(q * 16, 16)]
            dx = p - gxb
            far = (dx * dx) > thr
            lo_a = lo_a + jnp.where((p < gxb) & far, 1.0, 0.0)
            hi_a = hi_a + jnp.where((p > gxb) & far, 1.0, 0.0)
            return lo_a, hi_a

        lof, hif = lax.fori_loop(0, _N // 16, cnt_b, (zero, zero), unroll=8)
        lo_i = xreduce(lof, 0.0, jnp.add).astype(jnp.int32)
        hi_i = _N - xreduce(hif, 0.0, jnp.add).astype(jnp.int32)
        blo = jnp.clip(lo_i // 16, 0, _N // 16)
        bhi = jnp.clip((hi_i + 15) // 16, blo, _N // 16)

        init = (big,) * (3 * _GV) + (zero,) * (3 * _GV)

        def cand_body(nb, st):
            d1 = list(st[0:_GV])
            d2 = list(st[_GV:2 * _GV])
            d3 = list(st[2 * _GV:3 * _GV])
            f1 = list(st[3 * _GV:4 * _GV])
            f2 = list(st[4 * _GV:5 * _GV])
            f3 = list(st[5 * _GV:6 * _GV])
            base = nb * 16
            px16 = cand_v[0, pl.ds(base, 16)]
            py16 = cand_v[1, pl.ds(base, 16)]
            k216 = cand_v[2, pl.ds(base, 16)]
            ft16 = cand_v[3, pl.ds(base, 16)]
            for j in range(16):
                px = px16[j]
                py = py16[j]
                k2 = k216[j]
                ft = ft16[j]
                s = k2 - gx2 * px
                t = py + py
                for v in range(_GV):
                    d = (gub[v] + s) - t * gyb[v]
                    c1 = d < d1[v]
                    c2 = d < d2[v]
                    c3 = d < d3[v]
                    d3[v] = jnp.where(c3, jnp.where(c2, d2[v], d), d3[v])
                    f3[v] = jnp.where(c3, jnp.where(c2, f2[v], ft), f3[v])
                    d2[v] = jnp.where(c2, jnp.where(c1, d1[v], d), d2[v])
                    f2[v] = jnp.where(c2, jnp.where(c1, f1[v], ft), f2[v])
                    d1[v] = jnp.where(c1, d, d1[v])
                    f1[v] = jnp.where(c1, ft, f1[v])
            return tuple(d1 + d2 + d3 + f1 + f2 + f3)

        fin = lax.fori_loop(blo, bhi, cand_body, init)
        for s_i in range(3):
            for v in range(_GV):
                res_v[s_i, pl.ds(ci * _CH + v * 16, 16)] = (
                    fin[s_i * _GV + v] + mu[v])
                res_v[3 + s_i, pl.ds(ci * _CH + v * 16, 16)] = (
                    fin[(3 + s_i) * _GV + v])
        return 0

    lax.fori_loop(0, _NCHUNK, chunk_body, 0)
    for s_i in range(6):
        pltpu.sync_copy(res_v.at[s_i],
                        out_hbm.at[cloud, s_i, pl.ds(chunk * _PPW, _PPW)])


def _sc_knn(cand):
    fn = functools.partial(
        pl.kernel,
        mesh=plsc.VectorSubcoreMesh(core_axis_name="c", subcore_axis_name="s"),
        out_type=jax.ShapeDtypeStruct((2, 6, _M), jnp.float32),
        scratch_types=[
            pltpu.VMEM((5, _N), jnp.float32),
            pltpu.VMEM((6, _PPW), jnp.float32),
            pltpu.VMEM((32,), jnp.float32),
        ],
    )(_sc_knn_body)
    return fn(cand)


def _tc_loss_body(knn_ref, cosb_ref, wf_ref, rwf_ref, out_ref):
    def grid_interp(c):
        r1 = jnp.sqrt(jnp.maximum(knn_ref[c, 0], 1e-12))
        r2 = jnp.sqrt(jnp.maximum(knn_ref[c, 1], 1e-12))
        r3 = jnp.sqrt(jnp.maximum(knn_ref[c, 2], 1e-12))
        den = (r1 + r2) + r3
        w1 = r1 / den
        w2 = r2 / den
        w3 = r3 / den
        return (knn_ref[c, 3] * w1 + knn_ref[c, 4] * w2) + knn_ref[c, 5] * w3

    wb = wf_ref[...].astype(jnp.bfloat16).astype(jnp.float32)

    def coeffs(x):
        f = lax.dot_general(x, cosb_ref[...], (((1,), (0,)), ((), ())),
                            precision=lax.Precision.HIGHEST)
        ft = f.T.astype(jnp.bfloat16).astype(jnp.float32)
        fsel = jnp.reshape(
            jnp.broadcast_to(ft[:, None, :], (_MMAX, _LMAX, _NLAT)),
            (_MMAX * _LMAX, _NLAT))
        return jnp.sum(wb * fsel, axis=1, keepdims=True)

    dd = coeffs(grid_interp(0)) - coeffs(grid_interp(1))
    out_ref[0, 0] = jnp.sum(dd * dd * rwf_ref[...])


_tc_loss = pl.pallas_call(
    _tc_loss_body,
    out_shape=jax.ShapeDtypeStruct((1, 1), jnp.float32),
    out_specs=pl.BlockSpec(memory_space=pltpu.SMEM),
)


def kernel(pred, target):
    def prep(coords):
        r, sph = _to_spherical(coords)
        sph = sph.at[:, :, 1].add(-math.pi)
        px = sph[0, :, 0].astype(jnp.bfloat16).astype(jnp.float32)
        py = sph[0, :, 1].astype(jnp.bfloat16).astype(jnp.float32)
        k2 = jnp.sum(sph[0] ** 2, axis=-1)
        order = jnp.argsort(px)
        etamin = jnp.min(k2 - (px * px + py * py))
        return jnp.stack([px[order], py[order], k2[order], r[0, 0][order],
                          jnp.full((_N,), etamin)], axis=0)

    cand = jnp.stack([prep(pred), prep(target)], axis=0)
    knn = _sc_knn(cand)
    knn4 = knn.reshape(2, 6, _NLAT, _NLON)
    loss = _tc_loss(knn4, _COSB, _WF, _RWF)
    return loss[0, 0]
